# baseline (speedup 1.0000x reference)
import functools

import jax
import jax.numpy as jnp
from jax import lax
from jax.experimental import pallas as pl
from jax.experimental.pallas import tpu as pltpu
from jax.experimental.pallas import tpu_sc as plsc

NUM_FIELDS = 26
VOCAB = 100000
EMBED_DIM = 64
BATCH = 4096
LANES = 16
NSLOT = 2
ROWS = BATCH // 32


def _issue_row(tab_ref, idx_v, buf_v, sems, r, slot):
    descs = []
    vec0 = idx_v[r, pl.ds(0, LANES)]
    vec1 = idx_v[r, pl.ds(LANES, LANES)]
    for f in range(NUM_FIELDS):
        v = vec0[f] if f < LANES else vec1[f - LANES]
        descs.append(pltpu.async_copy(
            tab_ref.at[v], buf_v.at[slot, f], sems.at[slot]))
    return descs


def _accum_row(acc_v, buf_v, r, slot):
    for c in range(EMBED_DIM // LANES):
        d16 = lax.iota(jnp.int32, LANES) + c * LANES
        s = plsc.load_gather(buf_v.at[slot, 0],
                             [d16, jnp.zeros((LANES,), jnp.int32)])
        for f in range(1, NUM_FIELDS):
            s = s + plsc.load_gather(
                buf_v.at[slot, f], [d16, jnp.full((LANES,), f, jnp.int32)])
        acc_v[r, pl.ds(c * LANES, LANES)] = s


def _sc_body(tab_ref, x_ref, out_ref, idx_v, buf_v, acc_v, sems):
    nc = 2
    wid = lax.axis_index("s") * nc + lax.axis_index("c")
    base = wid * ROWS

    pltpu.sync_copy(x_ref.at[pl.ds(base, ROWS), :], idx_v)

    descs = {}
    for r in range(NSLOT):
        descs[r] = _issue_row(tab_ref, idx_v, buf_v, sems, r, r)

    def group(g, carry):
        for j in range(NSLOT):
            r = g * NSLOT + j
            for d in descs[j]:
                d.wait()
            _accum_row(acc_v, buf_v, r, j)
            nxt = r + NSLOT

            @pl.when(nxt < ROWS)
            def _():
                _issue_row(tab_ref, idx_v, buf_v, sems, nxt, j)
        return carry

    lax.fori_loop(0, ROWS // NSLOT, group, 0)
    pltpu.sync_copy(acc_v, out_ref.at[pl.ds(base, ROWS)])


def kernel(x, tables):
    xp = jnp.pad(x.astype(jnp.int32), ((0, 0), (0, 6)))
    vtab = tables.transpose(1, 2, 0)
    mesh = plsc.VectorSubcoreMesh(core_axis_name="c", subcore_axis_name="s")
    run = functools.partial(
        pl.kernel,
        mesh=mesh,
        out_type=jax.ShapeDtypeStruct((BATCH, EMBED_DIM), jnp.float32),
        scratch_types=[
            pltpu.VMEM((ROWS, 32), jnp.int32),
            pltpu.VMEM((NSLOT, NUM_FIELDS, EMBED_DIM, NUM_FIELDS),
                       jnp.float32),
            pltpu.VMEM((ROWS, EMBED_DIM), jnp.float32),
            pltpu.SemaphoreType.DMA((NSLOT,)),
        ],
        compiler_params=pltpu.CompilerParams(
            use_tc_tiling_on_sc=False, needs_layout_passes=False),
    )(_sc_body)
    return run(vtab, xp)


# speedup vs baseline: 3.3549x; 3.3549x over previous
import functools

import jax
import jax.numpy as jnp
from jax import lax
from jax.experimental import pallas as pl
from jax.experimental.pallas import tpu as pltpu
from jax.experimental.pallas import tpu_sc as plsc

NUM_FIELDS = 26
VOCAB = 100000
EMBED_DIM = 64
BATCH = 4096
LANES = 16
NBUF = 4
ROWS = BATCH // 32
PADD = 128
GROUPS = ((0, 7), (7, 14), (14, 20), (20, 26))


def _tab_at(tabs, f):
    for g, (lo, hi) in enumerate(GROUPS):
        if lo <= f < hi:
            return tabs[g].at[f - lo]
    raise ValueError(f)


def _sc_body(t0, t1, t2, t3, xt_ref, out_ref, idx_v, buf_v, acc_v, sems):
    tabs = (t0, t1, t2, t3)
    nc = 2
    wid = lax.axis_index("s") * nc + lax.axis_index("c")
    base = wid * ROWS

    pltpu.sync_copy(xt_ref.at[:, pl.ds(base, ROWS)], idx_v)

    descs = {}
    for f in range(NBUF):
        descs[f] = pltpu.async_copy(
            _tab_at(tabs, f).at[idx_v.at[f]], buf_v.at[f], sems.at[f])

    for f in range(NUM_FIELDS):
        slot = f % NBUF
        descs[f].wait()

        def accum(r, carry, slot=slot, f=f):
            for c in range(EMBED_DIM // LANES):
                sl = pl.ds(c * LANES, LANES)
                if f == 0:
                    acc_v[r, sl] = buf_v[slot, r, sl]
                else:
                    plsc.addupdate(acc_v.at[r, sl], buf_v[slot, r, sl])
            return carry

        lax.fori_loop(0, ROWS, accum, 0)

        nxt = f + NBUF
        if nxt < NUM_FIELDS:
            descs[nxt] = pltpu.async_copy(
                _tab_at(tabs, nxt).at[idx_v.at[nxt]], buf_v.at[slot],
                sems.at[slot])

    pltpu.sync_copy(acc_v, out_ref.at[pl.ds(base, ROWS)])


def kernel(x, tables):
    xt = jnp.pad(x.astype(jnp.int32), ((0, 0), (0, 6))).T
    tabs = [
        jnp.pad(tables[lo:hi], ((0, 0), (0, 0), (0, PADD - EMBED_DIM)))
        for lo, hi in GROUPS
    ]
    mesh = plsc.VectorSubcoreMesh(core_axis_name="c", subcore_axis_name="s")
    run = functools.partial(
        pl.kernel,
        mesh=mesh,
        out_type=jax.ShapeDtypeStruct((BATCH, PADD), jnp.float32),
        scratch_types=[
            pltpu.VMEM((32, ROWS), jnp.int32),
            pltpu.VMEM((NBUF, ROWS, PADD), jnp.float32),
            pltpu.VMEM((ROWS, PADD), jnp.float32),
            pltpu.SemaphoreType.DMA((NBUF,)),
        ],
        compiler_params=pltpu.CompilerParams(use_tc_tiling_on_sc=True),
    )(_sc_body)
    return run(*tabs, xt)[:, :EMBED_DIM]


# speedup vs baseline: 6.0597x; 1.8062x over previous
import functools

import jax
import jax.numpy as jnp
from jax import lax
from jax.experimental import pallas as pl
from jax.experimental.pallas import tpu as pltpu
from jax.experimental.pallas import tpu_sc as plsc

NUM_FIELDS = 26
VOCAB = 100000
EMBED_DIM = 64
BATCH = 4096
LANES = 16
NSLOT = 2
VROWS = VOCAB // LANES
ROWS = BATCH // 32


def _extract(vec0, vec1, f):
    return vec0[f] if f < LANES else vec1[f - LANES]


def _issue_row(tab_ref, idx_v, il_v, buf_v, sems, r, slot):
    descs = []
    vec0 = idx_v[r, pl.ds(0, LANES)]
    vec1 = idx_v[r, pl.ds(LANES, LANES)]
    d16 = lax.iota(jnp.int32, LANES) * VROWS
    for f in range(NUM_FIELDS):
        v = _extract(vec0, vec1, f)
        base = f * EMBED_DIM * VROWS + lax.shift_right_logical(v, 4)
        for c in range(EMBED_DIM // LANES):
            il_v[slot, f, pl.ds(c * LANES, LANES)] = (
                d16 + (base + c * LANES * VROWS))
        descs.append(pltpu.async_copy(
            tab_ref.at[il_v.at[slot, f]], buf_v.at[slot, f], sems.at[slot]))
    return descs


def _accum_row(acc_v, idx_v, buf_v, r, slot):
    vec0 = idx_v[r, pl.ds(0, LANES)]
    vec1 = idx_v[r, pl.ds(LANES, LANES)]
    for c in range(EMBED_DIM // LANES):
        d16 = lax.iota(jnp.int32, LANES) + c * LANES
        s = None
        for f in range(NUM_FIELDS):
            col = jnp.full((LANES,), 0, jnp.int32) + lax.bitwise_and(
                _extract(vec0, vec1, f), 15)
            g = plsc.load_gather(buf_v.at[slot, f], [d16, col])
            s = g if s is None else s + g
        acc_v[r, pl.ds(c * LANES, LANES)] = s


def _sc_body(tab_ref, x_ref, out_ref, idx_v, il_v, buf_v, acc_v, sems):
    nc = 2
    wid = lax.axis_index("s") * nc + lax.axis_index("c")
    base = wid * ROWS

    pltpu.sync_copy(x_ref.at[pl.ds(base, ROWS), :], idx_v)

    descs = {}
    for r in range(NSLOT):
        descs[r] = _issue_row(tab_ref, idx_v, il_v, buf_v, sems, r, r)

    def group(g, carry):
        for j in range(NSLOT):
            r = g * NSLOT + j
            for d in descs[j]:
                d.wait()
            _accum_row(acc_v, idx_v, buf_v, r, j)
            nxt = r + NSLOT

            @pl.when(nxt < ROWS)
            def _():
                _issue_row(tab_ref, idx_v, il_v, buf_v, sems, nxt, j)
        return carry

    lax.fori_loop(0, ROWS // NSLOT, group, 0)
    pltpu.sync_copy(acc_v, out_ref.at[pl.ds(base, ROWS)])


def kernel(x, tables):
    xp = jnp.pad(x.astype(jnp.int32), ((0, 0), (0, 6)))
    flat16 = tables.transpose(0, 2, 1).reshape(
        NUM_FIELDS * EMBED_DIM * VROWS, LANES)
    mesh = plsc.VectorSubcoreMesh(core_axis_name="c", subcore_axis_name="s")
    run = functools.partial(
        pl.kernel,
        mesh=mesh,
        out_type=jax.ShapeDtypeStruct((BATCH, EMBED_DIM), jnp.float32),
        scratch_types=[
            pltpu.VMEM((ROWS, 32), jnp.int32),
            pltpu.VMEM((NSLOT, NUM_FIELDS, EMBED_DIM), jnp.int32),
            pltpu.VMEM((NSLOT, NUM_FIELDS, EMBED_DIM, LANES), jnp.float32),
            pltpu.VMEM((ROWS, EMBED_DIM), jnp.float32),
            pltpu.SemaphoreType.DMA((NSLOT,)),
        ],
        compiler_params=pltpu.CompilerParams(
            use_tc_tiling_on_sc=False, needs_layout_passes=False),
    )(_sc_body)
    return run(flat16, xp)


# speedup vs baseline: 6.0670x; 1.0012x over previous
import functools

import jax
import jax.numpy as jnp
from jax import lax
from jax.experimental import pallas as pl
from jax.experimental.pallas import tpu as pltpu
from jax.experimental.pallas import tpu_sc as plsc

NUM_FIELDS = 26
VOCAB = 100000
EMBED_DIM = 64
BATCH = 4096
LANES = 16
NSLOT = 4
VROWS = VOCAB // LANES
ROWS = BATCH // 32


def _extract(vec0, vec1, f):
    return vec0[f] if f < LANES else vec1[f - LANES]


def _issue_row(tab_ref, idx_v, il_v, buf_v, sems, r, slot):
    descs = []
    vec0 = idx_v[r, pl.ds(0, LANES)]
    vec1 = idx_v[r, pl.ds(LANES, LANES)]
    d16 = lax.iota(jnp.int32, LANES) * VROWS
    for f in range(NUM_FIELDS):
        v = _extract(vec0, vec1, f)
        base = f * EMBED_DIM * VROWS + lax.shift_right_logical(v, 4)
        for c in range(EMBED_DIM // LANES):
            il_v[slot, f, pl.ds(c * LANES, LANES)] = (
                d16 + (base + c * LANES * VROWS))
        descs.append(pltpu.async_copy(
            tab_ref.at[il_v.at[slot, f]], buf_v.at[slot, f], sems.at[slot]))
    return descs


def _accum_row(acc_v, idx_v, buf_v, r, slot):
    vec0 = idx_v[r, pl.ds(0, LANES)]
    vec1 = idx_v[r, pl.ds(LANES, LANES)]
    for c in range(EMBED_DIM // LANES):
        d16 = lax.iota(jnp.int32, LANES) + c * LANES
        s = None
        for f in range(NUM_FIELDS):
            col = jnp.full((LANES,), 0, jnp.int32) + lax.bitwise_and(
                _extract(vec0, vec1, f), 15)
            g = plsc.load_gather(buf_v.at[slot, f], [d16, col])
            s = g if s is None else s + g
        acc_v[r, pl.ds(c * LANES, LANES)] = s


def _sc_body(tab_ref, x_ref, out_ref, idx_v, il_v, buf_v, acc_v, sems):
    nc = 2
    wid = lax.axis_index("s") * nc + lax.axis_index("c")
    base = wid * ROWS

    pltpu.sync_copy(x_ref.at[pl.ds(base, ROWS), :], idx_v)

    descs = {}
    for r in range(NSLOT):
        descs[r] = _issue_row(tab_ref, idx_v, il_v, buf_v, sems, r, r)

    def group(g, carry):
        for j in range(NSLOT):
            r = g * NSLOT + j
            for d in descs[j]:
                d.wait()
            _accum_row(acc_v, idx_v, buf_v, r, j)
            nxt = r + NSLOT

            @pl.when(nxt < ROWS)
            def _():
                _issue_row(tab_ref, idx_v, il_v, buf_v, sems, nxt, j)
        return carry

    lax.fori_loop(0, ROWS // NSLOT, group, 0)
    pltpu.sync_copy(acc_v, out_ref.at[pl.ds(base, ROWS)])


def kernel(x, tables):
    xp = jnp.pad(x.astype(jnp.int32), ((0, 0), (0, 6)))
    flat16 = tables.transpose(0, 2, 1).reshape(
        NUM_FIELDS * EMBED_DIM * VROWS, LANES)
    mesh = plsc.VectorSubcoreMesh(core_axis_name="c", subcore_axis_name="s")
    run = functools.partial(
        pl.kernel,
        mesh=mesh,
        out_type=jax.ShapeDtypeStruct((BATCH, EMBED_DIM), jnp.float32),
        scratch_types=[
            pltpu.VMEM((ROWS, 32), jnp.int32),
            pltpu.VMEM((NSLOT, NUM_FIELDS, EMBED_DIM), jnp.int32),
            pltpu.VMEM((NSLOT, NUM_FIELDS, EMBED_DIM, LANES), jnp.float32),
            pltpu.VMEM((ROWS, EMBED_DIM), jnp.float32),
            pltpu.SemaphoreType.DMA((NSLOT,)),
        ],
        compiler_params=pltpu.CompilerParams(
            use_tc_tiling_on_sc=False, needs_layout_passes=False),
    )(_sc_body)
    return run(flat16, xp)
